# baseline (speedup 1.0000x reference)
import functools

import jax
import jax.numpy as jnp
from jax import lax
from jax.experimental import pallas as pl
from jax.experimental.pallas import tpu as pltpu
from jax.experimental.pallas import tpu_sc as plsc

_B = 16384
_D = 64
_NC = 2
_NS = 16
_NW = _NC * _NS
_RPW = _B // _NW
_CH = 256
_NCHUNK = _RPW // _CH


def _sc_scores_body(u_ids, p_ids, n_ids, user_t, item_t, ent_t, out,
                    uidx, pidx, nidx,
                    ub0, ipb0, epb0, inb0, enb0,
                    accs, scores, sem0):
    wid = lax.axis_index("s") * _NC + lax.axis_index("c")

    row0 = wid * _NCHUNK
    pltpu.sync_copy(u_ids.at[pl.ds(row0, _NCHUNK)], uidx)
    pltpu.sync_copy(p_ids.at[pl.ds(row0, _NCHUNK)], pidx)
    pltpu.sync_copy(n_ids.at[pl.ds(row0, _NCHUNK)], nidx)

    lane = lax.broadcasted_iota(jnp.int32, (16,), 0)

    def chunk_copies(c):
        ub, ipb, epb, inb, enb = (ub0, ipb0, epb0, inb0, enb0)
        sem = sem0
        return [
            pltpu.make_async_copy(user_t.at[uidx.at[c]], ub, sem),
            pltpu.make_async_copy(item_t.at[pidx.at[c]], ipb, sem),
            pltpu.make_async_copy(ent_t.at[pidx.at[c]], epb, sem),
            pltpu.make_async_copy(item_t.at[nidx.at[c]], inb, sem),
            pltpu.make_async_copy(ent_t.at[nidx.at[c]], enb, sem),
        ]

    for c in range(_NCHUNK):
        cps = chunk_copies(c)
        for cp in cps:
            cp.start()
        for cp in cps:
            cp.wait()
        ub, ipb, epb, inb, enb = (ub0, ipb0, epb0, inb0, enb0)

        def row_body(r, _):
            acc = jnp.zeros((16,), jnp.float32)
            for k in range(4):
                sl = pl.ds(k * 16, 16)
                u = ub[r, sl]
                ip = ipb[r, sl]
                ep = epb[r, sl]
                i_n = inb[r, sl]
                en = enb[r, sl]
                acc = acc + u * ((ip + ep) - (i_n + en))
            accs[pl.ds(r * 17, 16)] = acc
            return 0

        lax.fori_loop(0, _CH, row_body, 0)

        def group_body(g, _):
            base = (g * 16 + lane) * 17
            sc = jnp.zeros((16,), jnp.float32)
            for l in range(16):
                sc = sc + plsc.load_gather(accs, [base + l])
            scores[pl.ds(c * _CH + g * 16, 16)] = sc
            return 0

        lax.fori_loop(0, _CH // 16, group_body, 0)

    pltpu.sync_copy(scores, out.at[pl.ds(wid * _RPW, _RPW)])


_sc_scores = functools.partial(
    pl.kernel,
    out_type=jax.ShapeDtypeStruct((_B,), jnp.float32),
    mesh=plsc.VectorSubcoreMesh(core_axis_name="c", subcore_axis_name="s"),
    compiler_params=pltpu.CompilerParams(
        needs_layout_passes=False, use_tc_tiling_on_sc=False),
    scratch_types=[
        pltpu.VMEM((_NCHUNK, _CH), jnp.int32),
        pltpu.VMEM((_NCHUNK, _CH), jnp.int32),
        pltpu.VMEM((_NCHUNK, _CH), jnp.int32),
        pltpu.VMEM((_CH, _D), jnp.float32),
        pltpu.VMEM((_CH, _D), jnp.float32),
        pltpu.VMEM((_CH, _D), jnp.float32),
        pltpu.VMEM((_CH, _D), jnp.float32),
        pltpu.VMEM((_CH, _D), jnp.float32),
        pltpu.VMEM((_CH * 17,), jnp.float32),
        pltpu.VMEM((_RPW,), jnp.float32),
        pltpu.SemaphoreType.DMA,
    ],
)(_sc_scores_body)


def _logsig_sum_body(x_ref, o_ref):
    x = x_ref[...]
    o_ref[0, 0] = jnp.sum(jnp.minimum(x, 0.0)
                          - jnp.log(1.0 + jnp.exp(-jnp.abs(x))))


_logsig_sum = pl.pallas_call(
    _logsig_sum_body,
    out_shape=jax.ShapeDtypeStruct((1, 1), jnp.float32),
    out_specs=pl.BlockSpec(memory_space=pltpu.SMEM),
)


def kernel(data, name, user_emb_matrix, item_emb_matrix, ent_emb_matrix,
           Mr_matrix, rel_emb_matrix):
    del name, Mr_matrix, rel_emb_matrix
    ids = data.astype(jnp.int32)
    u_ids = ids[:, 0].reshape(_B // _CH, _CH)
    p_ids = ids[:, 1].reshape(_B // _CH, _CH)
    n_ids = ids[:, 2].reshape(_B // _CH, _CH)
    scores = _sc_scores(u_ids, p_ids, n_ids, user_emb_matrix,
                        item_emb_matrix, ent_emb_matrix)
    loss = _logsig_sum(scores.reshape(_B // _CH, _CH))
    return loss[0, 0]


# speedup vs baseline: 1.2012x; 1.2012x over previous
import functools

import jax
import jax.numpy as jnp
from jax import lax
from jax.experimental import pallas as pl
from jax.experimental.pallas import tpu as pltpu
from jax.experimental.pallas import tpu_sc as plsc

_B = 16384
_D = 64
_NC = 2
_NS = 16
_NW = _NC * _NS
_RPW = _B // _NW
_CH = 256
_NCHUNK = _RPW // _CH


def _sc_scores_body(u_ids, p_ids, n_ids, user_t, comb_t, out,
                    uidx, pidx, nidx,
                    ub0, pb0, nb0,
                    accs, scores, sem0):
    wid = lax.axis_index("s") * _NC + lax.axis_index("c")

    row0 = wid * _NCHUNK
    pltpu.sync_copy(u_ids.at[pl.ds(row0, _NCHUNK)], uidx)
    pltpu.sync_copy(p_ids.at[pl.ds(row0, _NCHUNK)], pidx)
    pltpu.sync_copy(n_ids.at[pl.ds(row0, _NCHUNK)], nidx)

    lane = lax.broadcasted_iota(jnp.int32, (16,), 0)

    def chunk_copies(c):
        return [
            pltpu.make_async_copy(user_t.at[uidx.at[c]], ub0, sem0),
            pltpu.make_async_copy(comb_t.at[pidx.at[c]], pb0, sem0),
            pltpu.make_async_copy(comb_t.at[nidx.at[c]], nb0, sem0),
        ]

    for c in range(_NCHUNK):
        cps = chunk_copies(c)
        for cp in cps:
            cp.start()
        for cp in cps:
            cp.wait()
        def row_body(r, _):
            acc = jnp.zeros((16,), jnp.float32)
            for k in range(4):
                sl = pl.ds(k * 16, 16)
                acc = acc + ub0[r, sl] * (pb0[r, sl] - nb0[r, sl])
            accs[pl.ds(r * 17, 16)] = acc
            return 0

        lax.fori_loop(0, _CH, row_body, 0)

        def group_body(g, _):
            base = (g * 16 + lane) * 17
            sc = jnp.zeros((16,), jnp.float32)
            for l in range(16):
                sc = sc + plsc.load_gather(accs, [base + l])
            scores[pl.ds(c * _CH + g * 16, 16)] = sc
            return 0

        lax.fori_loop(0, _CH // 16, group_body, 0)

    pltpu.sync_copy(scores, out.at[pl.ds(wid * _RPW, _RPW)])


_sc_scores = functools.partial(
    pl.kernel,
    out_type=jax.ShapeDtypeStruct((_B,), jnp.float32),
    mesh=plsc.VectorSubcoreMesh(core_axis_name="c", subcore_axis_name="s"),
    compiler_params=pltpu.CompilerParams(
        needs_layout_passes=False, use_tc_tiling_on_sc=False),
    scratch_types=[
        pltpu.VMEM((_NCHUNK, _CH), jnp.int32),
        pltpu.VMEM((_NCHUNK, _CH), jnp.int32),
        pltpu.VMEM((_NCHUNK, _CH), jnp.int32),
        pltpu.VMEM((_CH, _D), jnp.float32),
        pltpu.VMEM((_CH, _D), jnp.float32),
        pltpu.VMEM((_CH, _D), jnp.float32),
        pltpu.VMEM((_CH * 17,), jnp.float32),
        pltpu.VMEM((_RPW,), jnp.float32),
        pltpu.SemaphoreType.DMA,
    ],
)(_sc_scores_body)


def _logsig_sum_body(x_ref, o_ref):
    x = x_ref[...]
    o_ref[0, 0] = jnp.sum(jnp.minimum(x, 0.0)
                          - jnp.log(1.0 + jnp.exp(-jnp.abs(x))))


_logsig_sum = pl.pallas_call(
    _logsig_sum_body,
    out_shape=jax.ShapeDtypeStruct((1, 1), jnp.float32),
    out_specs=pl.BlockSpec(memory_space=pltpu.SMEM),
)


def kernel(data, name, user_emb_matrix, item_emb_matrix, ent_emb_matrix,
           Mr_matrix, rel_emb_matrix):
    del name, Mr_matrix, rel_emb_matrix
    ids = data.astype(jnp.int32)
    u_ids = ids[:, 0].reshape(_B // _CH, _CH)
    p_ids = ids[:, 1].reshape(_B // _CH, _CH)
    n_ids = ids[:, 2].reshape(_B // _CH, _CH)
    combined = item_emb_matrix + ent_emb_matrix
    scores = _sc_scores(u_ids, p_ids, n_ids, user_emb_matrix, combined)
    loss = _logsig_sum(scores.reshape(_B // _CH, _CH))
    return loss[0, 0]


# speedup vs baseline: 1.2427x; 1.0345x over previous
import functools

import jax
import jax.numpy as jnp
from jax import lax
from jax.experimental import pallas as pl
from jax.experimental.pallas import tpu as pltpu
from jax.experimental.pallas import tpu_sc as plsc

_B = 16384
_D = 64
_NC = 2
_NS = 16
_NW = _NC * _NS
_RPW = _B // _NW
_CH = 128
_DP = 128
_NCHUNK = _RPW // _CH


def _sc_scores_body(u_ids, p_ids, n_ids, user_t, comb_t, out,
                    uidx, pidx, nidx,
                    ub0, pb0, nb0,
                    accs, scores, sem0):
    wid = lax.axis_index("s") * _NC + lax.axis_index("c")

    base = wid * _RPW
    pltpu.sync_copy(u_ids.at[pl.ds(base, _RPW)], uidx)
    pltpu.sync_copy(p_ids.at[pl.ds(base, _RPW)], pidx)
    pltpu.sync_copy(n_ids.at[pl.ds(base, _RPW)], nidx)

    lane = lax.broadcasted_iota(jnp.int32, (16,), 0)

    def chunk_copies(c):
        sl = pl.ds(c * _CH, _CH)
        return [
            pltpu.make_async_copy(user_t.at[uidx.at[sl]], ub0, sem0),
            pltpu.make_async_copy(comb_t.at[pidx.at[sl]], pb0, sem0),
            pltpu.make_async_copy(comb_t.at[nidx.at[sl]], nb0, sem0),
        ]

    for c in range(_NCHUNK):
        cps = chunk_copies(c)
        for cp in cps:
            cp.start()
        for cp in cps:
            cp.wait()
        def row_body(r, _):
            acc = jnp.zeros((16,), jnp.float32)
            for k in range(4):
                sl = pl.ds(k * 16, 16)
                acc = acc + ub0[r, sl] * (pb0[r, sl] - nb0[r, sl])
            accs[pl.ds(r * 17, 16)] = acc
            return 0

        lax.fori_loop(0, _CH, row_body, 0)

        def group_body(g, _):
            base = (g * 16 + lane) * 17
            sc = jnp.zeros((16,), jnp.float32)
            for l in range(16):
                sc = sc + plsc.load_gather(accs, [base + l])
            scores[pl.ds(c * _CH + g * 16, 16)] = sc
            return 0

        lax.fori_loop(0, _CH // 16, group_body, 0)

    pltpu.sync_copy(scores, out.at[pl.ds(wid * _RPW, _RPW)])


_sc_scores = functools.partial(
    pl.kernel,
    out_type=jax.ShapeDtypeStruct((_B,), jnp.float32),
    mesh=plsc.VectorSubcoreMesh(core_axis_name="c", subcore_axis_name="s"),
    compiler_params=pltpu.CompilerParams(
        needs_layout_passes=False, use_tc_tiling_on_sc=True),
    scratch_types=[
        pltpu.VMEM((_RPW,), jnp.int32),
        pltpu.VMEM((_RPW,), jnp.int32),
        pltpu.VMEM((_RPW,), jnp.int32),
        pltpu.VMEM((_CH, _DP), jnp.float32),
        pltpu.VMEM((_CH, _DP), jnp.float32),
        pltpu.VMEM((_CH, _DP), jnp.float32),
        pltpu.VMEM((_CH * 17,), jnp.float32),
        pltpu.VMEM((_RPW,), jnp.float32),
        pltpu.SemaphoreType.DMA,
    ],
)(_sc_scores_body)


def _logsig_sum_body(x_ref, o_ref):
    x = x_ref[...]
    o_ref[0, 0] = jnp.sum(jnp.minimum(x, 0.0)
                          - jnp.log(1.0 + jnp.exp(-jnp.abs(x))))


_logsig_sum = pl.pallas_call(
    _logsig_sum_body,
    out_shape=jax.ShapeDtypeStruct((1, 1), jnp.float32),
    out_specs=pl.BlockSpec(memory_space=pltpu.SMEM),
)


def kernel(data, name, user_emb_matrix, item_emb_matrix, ent_emb_matrix,
           Mr_matrix, rel_emb_matrix):
    del name, Mr_matrix, rel_emb_matrix
    ids = data.astype(jnp.int32)
    u_ids = ids[:, 0]
    p_ids = ids[:, 1]
    n_ids = ids[:, 2]
    pad = ((0, 0), (0, _DP - _D))
    user128 = jnp.pad(user_emb_matrix, pad)
    comb128 = jnp.pad(item_emb_matrix + ent_emb_matrix, pad)
    scores = _sc_scores(u_ids, p_ids, n_ids, user128, comb128)
    loss = _logsig_sum(scores.reshape(_B // _CH, _CH))
    return loss[0, 0]
